# baseline (speedup 1.0000x reference)
import functools

import jax
import jax.numpy as jnp
from jax import lax
from jax.experimental import pallas as pl
from jax.experimental.pallas import tpu as pltpu
from jax.experimental.pallas import tpu_sc as plsc


def _bf16_rne(x):
    u = lax.bitcast_convert_type(x, jnp.int32)
    r = (u + jnp.int32(0x7FFF) + ((u >> 16) & 1)) & jnp.int32(-65536)
    return lax.bitcast_convert_type(r, jnp.float32)


def _sortable(x):
    xi = lax.bitcast_convert_type(x, jnp.int32)
    return xi ^ ((xi >> 31) & jnp.int32(0x7FFFFFFF))


def _topk_mask(v, s, k):
    tau = _kth_largest(v, k)
    gt = v > tau
    eq = v == tau
    need = k - jnp.sum(gt.astype(jnp.int32), axis=-1, keepdims=True)
    N = v.shape[-1]
    iota = lax.broadcasted_iota(jnp.int32, v.shape, v.ndim - 1)
    eqi = eq.astype(jnp.int32)
    c = jnp.zeros(v.shape[:-1] + (1,), jnp.int32)
    b = N // 2
    while b >= 1:
        cnt_lt = jnp.sum(jnp.where(iota < (c + b), eqi, 0),
                         axis=-1, keepdims=True)
        c = jnp.where(cnt_lt < need, c + b, c)
        b //= 2
    keep = gt | (eq & (iota <= c))
    return jnp.where(keep, s, 0.0)


def _kth_largest(v, k):
    p0 = jnp.full(v.shape[:-1] + (1,), jnp.int32(-(2**31)))

    def body(i, p):
        c = p + (jnp.int32(1) << (jnp.int32(31) - i))
        cnt = jnp.sum((v >= c).astype(jnp.int32), axis=-1, keepdims=True)
        return jnp.where(cnt >= k, c, p)

    return lax.fori_loop(0, 32, body, p0)


def _scores_body(values_ref, cw3_ref, prev_ref, query_ref, w1t_ref, w2t_ref,
                 qb_ref, locp_ref, vw_ref, vb_ref, out_ref):
    B, Tb, H = values_ref.shape
    prev = prev_ref[...]
    q = query_ref[...]
    Rb = min(16, B)
    for h in range(3):
        cw = cw3_ref[h]
        convo = lax.dot_general(prev, cw, (((1,), (1,)), ((), ())),
                                preferred_element_type=jnp.float32)
        qt = jnp.dot(q, w2t_ref[h],
                     preferred_element_type=jnp.float32) + qb_ref[h][None, :]
        vwb = jnp.broadcast_to(vw_ref[h][:, None], (H, 128))
        for rb in range(0, B, Rb):
            v = values_ref[rb:rb + Rb].reshape(Rb * Tb, H)
            p1 = jnp.dot(v, w1t_ref[h],
                         preferred_element_type=jnp.float32).reshape(Rb, Tb, H)
            s1 = (p1 + qt[rb:rb + Rb, None, :]
                  + convo[rb:rb + Rb, :, None] * locp_ref[h][None, None, :])
            z = jnp.tanh(s1).reshape(Rb * Tb, H)
            sc = jnp.dot(z, vwb,
                         preferred_element_type=jnp.float32)
            sc = sc.reshape(Rb, Tb, 128)[:, :, 0] + vb_ref[h]
            out_ref[h, rb:rb + Rb] = sc


def _scores_call(values, cw3, prev2, query, w1t, w2t, qb, locp, vw, vb):
    B, T, H = values.shape
    Tb = 128 if T % 128 == 0 else T
    return pl.pallas_call(
        _scores_body,
        grid=(T // Tb,),
        in_specs=[
            pl.BlockSpec((B, Tb, H), lambda i: (0, i, 0)),
            pl.BlockSpec((3, Tb, T), lambda i: (0, i, 0)),
            pl.BlockSpec((B, T), lambda i: (0, 0)),
            pl.BlockSpec((B, H), lambda i: (0, 0)),
            pl.BlockSpec((3, H, H), lambda i: (0, 0, 0)),
            pl.BlockSpec((3, H, H), lambda i: (0, 0, 0)),
            pl.BlockSpec((3, H), lambda i: (0, 0)),
            pl.BlockSpec((3, H), lambda i: (0, 0)),
            pl.BlockSpec((3, H), lambda i: (0, 0)),
            pl.BlockSpec(memory_space=pltpu.SMEM),
        ],
        out_specs=pl.BlockSpec((3, B, Tb), lambda i: (0, 0, i)),
        out_shape=jax.ShapeDtypeStruct((3, B, T), jnp.float32),
    )(values, cw3, prev2, query, w1t, w2t, qb, locp, vw, vb)


def _mask_body(s3_ref, wmix_ref, smask_ref, sig_ref, *, k):
    s3 = s3_ref[...]
    v3 = _sortable(s3)
    m3 = _topk_mask(v3, s3, k)
    m3b = _bf16_rne(m3)
    w0 = _bf16_rne(wmix_ref[0])
    w1 = _bf16_rne(wmix_ref[1])
    w2 = _bf16_rne(wmix_ref[2])
    cmb = m3b[0] * w0 + m3b[1] * w1 + m3b[2] * w2 + wmix_ref[3]
    vc = _sortable(cmb)
    sm = _topk_mask(vc, cmb, k)
    smask_ref[...] = sm
    sig_ref[...] = 1.0 / (1.0 + jnp.exp(-sm))


def _mask_call(scores3, wmix4, k):
    import functools
    _, B, T = scores3.shape
    return pl.pallas_call(
        functools.partial(_mask_body, k=k),
        in_specs=[
            pl.BlockSpec((3, B, T), lambda: (0, 0, 0)),
            pl.BlockSpec(memory_space=pltpu.SMEM),
        ],
        out_specs=[
            pl.BlockSpec((B, T), lambda: (0, 0)),
            pl.BlockSpec((B, T), lambda: (0, 0)),
        ],
        out_shape=[
            jax.ShapeDtypeStruct((B, T), jnp.float32),
            jax.ShapeDtypeStruct((B, T), jnp.float32),
        ],
    )(scores3, wmix4)


def _sc_sortable(v):
    u = lax.bitcast_convert_type(v, jnp.int32)
    return u ^ ((u >> 31) & jnp.int32(0x7FFFFFFF))


def _sc_mask_row(row_ref, cand_a, cand_b, k, T):
    nch_full = T // 16
    iota16 = lax.iota(jnp.int32, 16)

    def kb(j, acc):
        v = row_ref[pl.ds(j * 16, 16)]
        cand_a[pl.ds(j * 16, 16)] = _sc_sortable(v)
        return acc

    lax.fori_loop(0, nch_full, kb, jnp.int32(0))

    def half_step(i2, src, dst, p, rem, n):
        c = p + (jnp.int32(1) << (jnp.int32(31) - i2))
        nch = (n + 15) // 16

        def cb(j, acc):
            v = src[pl.ds(j * 16, 16)]
            valid = (j * 16 + iota16) < n
            m = (v >= c) & valid
            return acc + jnp.sum(jnp.where(m, 1, 0))

        cnt = lax.fori_loop(0, nch, cb, jnp.int32(0))
        takehi = cnt >= rem
        want = jnp.where(takehi, jnp.int32(1), jnp.int32(0))

        def pb(j, off):
            v = src[pl.ds(j * 16, 16)]
            valid = (j * 16 + iota16) < n
            m = (jnp.where(v >= c, 1, 0) == want) & valid
            plsc.store_compressed(dst.at[pl.ds(off, 16)], v, mask=m)
            return off + jnp.sum(jnp.where(m, 1, 0))

        lax.fori_loop(0, nch, pb, jnp.int32(0))
        p = jnp.where(takehi, c, p)
        rem = jnp.where(takehi, rem, rem - cnt)
        n = jnp.where(takehi, cnt, n - cnt)
        return p, rem, n

    def bit_pair(i, carry):
        p, rem, n = carry
        p, rem, n = half_step(2 * i, cand_a, cand_b, p, rem, n)
        p, rem, n = half_step(2 * i + 1, cand_b, cand_a, p, rem, n)
        return p, rem, n

    p, rem, _ = lax.fori_loop(
        0, 16, bit_pair,
        (jnp.int32(-(2**31)), jnp.int32(k), jnp.int32(T)))
    tau = p

    def fb(j, run):
        v = row_ref[pl.ds(j * 16, 16)]
        u = _sc_sortable(v)
        m_gt = u > tau
        m_eq = u == tau
        eqi = jnp.where(m_eq, 1, 0)
        pc = plsc.cumsum(eqi)
        keep = m_gt | (m_eq & ((run + pc) <= rem))
        row_ref[pl.ds(j * 16, 16)] = jnp.where(keep, v, 0.0)
        return run + jnp.sum(eqi)

    lax.fori_loop(0, nch_full, fb, jnp.int32(0))


def _sc_rne(x):
    u = lax.bitcast_convert_type(x, jnp.int32)
    r = (u + jnp.int32(0x7FFF) + ((u >> 16) & 1)) & jnp.int32(-65536)
    return lax.bitcast_convert_type(r, jnp.float32)


def _sc_mask_call(scores3, wsplat, k):
    _, B, T = scores3.shape
    info = plsc.get_sparse_core_info()
    nw = info.num_cores * info.num_subcores
    rows_per = B // nw

    @functools.partial(
        pl.kernel,
        mesh=plsc.VectorSubcoreMesh(core_axis_name="c", subcore_axis_name="s"),
        compiler_params=pltpu.CompilerParams(needs_layout_passes=False),
        out_type=[
            jax.ShapeDtypeStruct((B, T), jnp.float32),
            jax.ShapeDtypeStruct((B, T), jnp.float32),
        ],
        scratch_types=[
            pltpu.VMEM((T,), jnp.float32),
            pltpu.VMEM((T,), jnp.float32),
            pltpu.VMEM((T,), jnp.float32),
            pltpu.VMEM((T,), jnp.float32),
            pltpu.VMEM((T + 16,), jnp.int32),
            pltpu.VMEM((T + 16,), jnp.int32),
            pltpu.VMEM((64,), jnp.float32),
        ],
    )
    def body(s3_hbm, w_hbm, smask_hbm, sig_hbm,
             row0, row1, row2, cmb, cand_a, cand_b, wv):
        wid = lax.axis_index("s") * info.num_cores + lax.axis_index("c")
        pltpu.sync_copy(w_hbm, wv)
        w0 = _sc_rne(wv[pl.ds(0, 16)])
        w1 = _sc_rne(wv[pl.ds(16, 16)])
        w2 = _sc_rne(wv[pl.ds(32, 16)])
        wb = wv[pl.ds(48, 16)]
        for r in range(rows_per):
            b = wid * rows_per + r
            pltpu.sync_copy(s3_hbm.at[0, b], row0)
            pltpu.sync_copy(s3_hbm.at[1, b], row1)
            pltpu.sync_copy(s3_hbm.at[2, b], row2)
            for rr in (row0, row1, row2):
                _sc_mask_row(rr, cand_a, cand_b, k, T)

            def mix(j, acc):
                m0 = _sc_rne(row0[pl.ds(j * 16, 16)])
                m1 = _sc_rne(row1[pl.ds(j * 16, 16)])
                m2 = _sc_rne(row2[pl.ds(j * 16, 16)])
                cmb[pl.ds(j * 16, 16)] = m0 * w0 + m1 * w1 + m2 * w2 + wb
                return acc

            lax.fori_loop(0, T // 16, mix, jnp.int32(0))
            _sc_mask_row(cmb, cand_a, cand_b, k, T)
            pltpu.sync_copy(cmb, smask_hbm.at[b])

            def sg(j, acc):
                x = cmb[pl.ds(j * 16, 16)]
                row0[pl.ds(j * 16, 16)] = 1.0 / (1.0 + jnp.exp(-x))
                return acc

            lax.fori_loop(0, T // 16, sg, jnp.int32(0))
            pltpu.sync_copy(row0, sig_hbm.at[b])

    return body(scores3, wsplat)


def _finalize_body(values_ref, sig_ref, ctx_ref, att_ref):
    sg = sig_ref[...]
    colsum = jnp.sum(sg, axis=0, keepdims=True)
    att = sg / colsum
    att_ref[...] = att
    v = values_ref[...]
    partial = jnp.sum(att[:, :, None] * v, axis=1)

    @pl.when(pl.program_id(0) == 0)
    def _():
        ctx_ref[...] = jnp.zeros_like(ctx_ref)

    ctx_ref[...] += partial


def _finalize_call(values, sig):
    B, T, H = values.shape
    Tb = 128 if T % 128 == 0 else T
    return pl.pallas_call(
        _finalize_body,
        grid=(T // Tb,),
        in_specs=[
            pl.BlockSpec((B, Tb, H), lambda i: (0, i, 0)),
            pl.BlockSpec((B, Tb), lambda i: (0, i)),
        ],
        out_specs=[
            pl.BlockSpec((B, H), lambda i: (0, 0)),
            pl.BlockSpec((B, Tb), lambda i: (0, i)),
        ],
        out_shape=[
            jax.ShapeDtypeStruct((B, H), jnp.float32),
            jax.ShapeDtypeStruct((B, T), jnp.float32),
        ],
    )(values, sig)


def kernel(query, values, prev_att, params):
    B, T, H = values.shape
    heads = params['heads']
    mid = heads[0]['conv_w'].shape[-1] // 2
    k = T * 2 // 3

    prev2 = prev_att[..., 0]
    cw3 = jnp.stack([hp['conv_w'][:, :, mid] for hp in heads])
    w1t = jnp.stack([hp['W1_w'].T for hp in heads])
    w2t = jnp.stack([hp['W2_w'].T for hp in heads])
    qb = jnp.stack([hp['W1_b'] + hp['W2_b'] for hp in heads])
    locp = jnp.stack([hp['loc_proj_w'][:, 0] for hp in heads])
    vw = jnp.stack([hp['V_w'][0] for hp in heads])
    vb = jnp.stack([hp['V_b'][0] for hp in heads])
    ws = [jnp.full((16,), params['W_w'][0, i], jnp.float32) for i in range(3)]
    wsplat = jnp.concatenate(ws + [jnp.full((16,), params['W_b'][0],
                                            jnp.float32)])

    scores3 = _scores_call(values, cw3, prev2, query, w1t, w2t, qb, locp,
                           vw, vb)
    smask, sig = _sc_mask_call(scores3, wsplat, k)
    ctx, att = _finalize_call(values, sig)
    return ctx, att[..., None], smask[..., None]


# speedup vs baseline: 1.0073x; 1.0073x over previous
import functools

import jax
import jax.numpy as jnp
from jax import lax
from jax.experimental import pallas as pl
from jax.experimental.pallas import tpu as pltpu
from jax.experimental.pallas import tpu_sc as plsc


def _bf16_rne(x):
    u = lax.bitcast_convert_type(x, jnp.int32)
    r = (u + jnp.int32(0x7FFF) + ((u >> 16) & 1)) & jnp.int32(-65536)
    return lax.bitcast_convert_type(r, jnp.float32)


def _sortable(x):
    xi = lax.bitcast_convert_type(x, jnp.int32)
    return xi ^ ((xi >> 31) & jnp.int32(0x7FFFFFFF))


def _topk_mask(v, s, k):
    tau = _kth_largest(v, k)
    gt = v > tau
    eq = v == tau
    need = k - jnp.sum(gt.astype(jnp.int32), axis=-1, keepdims=True)
    N = v.shape[-1]
    iota = lax.broadcasted_iota(jnp.int32, v.shape, v.ndim - 1)
    eqi = eq.astype(jnp.int32)
    c = jnp.zeros(v.shape[:-1] + (1,), jnp.int32)
    b = N // 2
    while b >= 1:
        cnt_lt = jnp.sum(jnp.where(iota < (c + b), eqi, 0),
                         axis=-1, keepdims=True)
        c = jnp.where(cnt_lt < need, c + b, c)
        b //= 2
    keep = gt | (eq & (iota <= c))
    return jnp.where(keep, s, 0.0)


def _kth_largest(v, k):
    p0 = jnp.full(v.shape[:-1] + (1,), jnp.int32(-(2**31)))

    def body(i, p):
        c = p + (jnp.int32(1) << (jnp.int32(31) - i))
        cnt = jnp.sum((v >= c).astype(jnp.int32), axis=-1, keepdims=True)
        return jnp.where(cnt >= k, c, p)

    return lax.fori_loop(0, 32, body, p0)


def _scores_body(values_ref, cw3_ref, prev_ref, query_ref, w1t_ref, w2t_ref,
                 qb_ref, locp_ref, vw_ref, vb_ref, out_ref):
    B, Tb, H = values_ref.shape
    prev = prev_ref[...]
    q = query_ref[...]
    Rb = min(16, B)
    for h in range(3):
        cw = cw3_ref[h]
        convo = lax.dot_general(prev, cw, (((1,), (1,)), ((), ())),
                                preferred_element_type=jnp.float32)
        qt = jnp.dot(q, w2t_ref[h],
                     preferred_element_type=jnp.float32) + qb_ref[h][None, :]
        vwb = jnp.broadcast_to(vw_ref[h][:, None], (H, 128))
        for rb in range(0, B, Rb):
            v = values_ref[rb:rb + Rb].reshape(Rb * Tb, H)
            p1 = jnp.dot(v, w1t_ref[h],
                         preferred_element_type=jnp.float32).reshape(Rb, Tb, H)
            s1 = (p1 + qt[rb:rb + Rb, None, :]
                  + convo[rb:rb + Rb, :, None] * locp_ref[h][None, None, :])
            z = jnp.tanh(s1).reshape(Rb * Tb, H)
            sc = jnp.dot(z, vwb,
                         preferred_element_type=jnp.float32)
            sc = sc.reshape(Rb, Tb, 128)[:, :, 0] + vb_ref[h]
            out_ref[h, rb:rb + Rb] = sc


def _scores_call(values, cw3, prev2, query, w1t, w2t, qb, locp, vw, vb):
    B, T, H = values.shape
    Tb = 128 if T % 128 == 0 else T
    return pl.pallas_call(
        _scores_body,
        grid=(T // Tb,),
        in_specs=[
            pl.BlockSpec((B, Tb, H), lambda i: (0, i, 0)),
            pl.BlockSpec((3, Tb, T), lambda i: (0, i, 0)),
            pl.BlockSpec((B, T), lambda i: (0, 0)),
            pl.BlockSpec((B, H), lambda i: (0, 0)),
            pl.BlockSpec((3, H, H), lambda i: (0, 0, 0)),
            pl.BlockSpec((3, H, H), lambda i: (0, 0, 0)),
            pl.BlockSpec((3, H), lambda i: (0, 0)),
            pl.BlockSpec((3, H), lambda i: (0, 0)),
            pl.BlockSpec((3, H), lambda i: (0, 0)),
            pl.BlockSpec(memory_space=pltpu.SMEM),
        ],
        out_specs=pl.BlockSpec((3, B, Tb), lambda i: (0, 0, i)),
        out_shape=jax.ShapeDtypeStruct((3, B, T), jnp.float32),
    )(values, cw3, prev2, query, w1t, w2t, qb, locp, vw, vb)


def _mask_body(s3_ref, wmix_ref, smask_ref, sig_ref, *, k):
    s3 = s3_ref[...]
    v3 = _sortable(s3)
    m3 = _topk_mask(v3, s3, k)
    m3b = _bf16_rne(m3)
    w0 = _bf16_rne(wmix_ref[0])
    w1 = _bf16_rne(wmix_ref[1])
    w2 = _bf16_rne(wmix_ref[2])
    cmb = m3b[0] * w0 + m3b[1] * w1 + m3b[2] * w2 + wmix_ref[3]
    vc = _sortable(cmb)
    sm = _topk_mask(vc, cmb, k)
    smask_ref[...] = sm
    sig_ref[...] = 1.0 / (1.0 + jnp.exp(-sm))


def _mask_call(scores3, wmix4, k):
    import functools
    _, B, T = scores3.shape
    return pl.pallas_call(
        functools.partial(_mask_body, k=k),
        in_specs=[
            pl.BlockSpec((3, B, T), lambda: (0, 0, 0)),
            pl.BlockSpec(memory_space=pltpu.SMEM),
        ],
        out_specs=[
            pl.BlockSpec((B, T), lambda: (0, 0)),
            pl.BlockSpec((B, T), lambda: (0, 0)),
        ],
        out_shape=[
            jax.ShapeDtypeStruct((B, T), jnp.float32),
            jax.ShapeDtypeStruct((B, T), jnp.float32),
        ],
    )(scores3, wmix4)


def _sc_sortable(v):
    u = lax.bitcast_convert_type(v, jnp.int32)
    return u ^ ((u >> 31) & jnp.int32(0x7FFFFFFF))


def _sc_mask_row(row_ref, cand_a, cand_b, k, T):
    U = 4
    iota16 = lax.iota(jnp.int32, 16)

    def kb(j, acc):
        for u in range(U):
            v = row_ref[pl.ds((j * U + u) * 16, 16)]
            cand_a[pl.ds((j * U + u) * 16, 16)] = _sc_sortable(v)
        return acc

    lax.fori_loop(0, T // (16 * U), kb, jnp.int32(0))

    def half_step(i2, src, dst, p, rem, n):
        c = p + (jnp.int32(1) << (jnp.int32(31) - i2))
        nch = (n + (16 * U - 1)) // (16 * U)

        def cb(j, acc):
            for u in range(U):
                v = src[pl.ds((j * U + u) * 16, 16)]
                valid = ((j * U + u) * 16 + iota16) < n
                m = (v >= c) & valid
                acc = acc + jnp.sum(jnp.where(m, 1, 0))
            return acc

        cnt = lax.fori_loop(0, nch, cb, jnp.int32(0))
        takehi = cnt >= rem
        want = jnp.where(takehi, jnp.int32(1), jnp.int32(0))

        def pb(j, off):
            for u in range(U):
                v = src[pl.ds((j * U + u) * 16, 16)]
                valid = ((j * U + u) * 16 + iota16) < n
                m = (jnp.where(v >= c, 1, 0) == want) & valid
                plsc.store_compressed(dst.at[pl.ds(off, 16)], v, mask=m)
                off = off + jnp.sum(jnp.where(m, 1, 0))
            return off

        lax.fori_loop(0, nch, pb, jnp.int32(0))
        p = jnp.where(takehi, c, p)
        rem = jnp.where(takehi, rem, rem - cnt)
        n = jnp.where(takehi, cnt, n - cnt)
        return p, rem, n

    def bit_pair(i, carry):
        p, rem, n = carry
        p, rem, n = half_step(2 * i, cand_a, cand_b, p, rem, n)
        p, rem, n = half_step(2 * i + 1, cand_b, cand_a, p, rem, n)
        return p, rem, n

    p, rem, _ = lax.fori_loop(
        0, 16, bit_pair,
        (jnp.int32(-(2**31)), jnp.int32(k), jnp.int32(T)))
    tau = p

    def fb(j, run):
        for u in range(U):
            v = row_ref[pl.ds((j * U + u) * 16, 16)]
            uu = _sc_sortable(v)
            m_gt = uu > tau
            m_eq = uu == tau
            eqi = jnp.where(m_eq, 1, 0)
            pc = plsc.cumsum(eqi)
            keep = m_gt | (m_eq & ((run + pc) <= rem))
            row_ref[pl.ds((j * U + u) * 16, 16)] = jnp.where(keep, v, 0.0)
            run = run + jnp.sum(eqi)
        return run

    lax.fori_loop(0, T // (16 * U), fb, jnp.int32(0))


def _sc_rne(x):
    u = lax.bitcast_convert_type(x, jnp.int32)
    r = (u + jnp.int32(0x7FFF) + ((u >> 16) & 1)) & jnp.int32(-65536)
    return lax.bitcast_convert_type(r, jnp.float32)


def _sc_mask_call(scores3, wsplat, k):
    _, B, T = scores3.shape
    info = plsc.get_sparse_core_info()
    nw = info.num_cores * info.num_subcores
    rows_per = B // nw

    @functools.partial(
        pl.kernel,
        mesh=plsc.VectorSubcoreMesh(core_axis_name="c", subcore_axis_name="s"),
        compiler_params=pltpu.CompilerParams(needs_layout_passes=False),
        out_type=[
            jax.ShapeDtypeStruct((B, T), jnp.float32),
            jax.ShapeDtypeStruct((B, T), jnp.float32),
        ],
        scratch_types=[
            pltpu.VMEM((T,), jnp.float32),
            pltpu.VMEM((T,), jnp.float32),
            pltpu.VMEM((T,), jnp.float32),
            pltpu.VMEM((T,), jnp.float32),
            pltpu.VMEM((T + 16,), jnp.int32),
            pltpu.VMEM((T + 16,), jnp.int32),
            pltpu.VMEM((64,), jnp.float32),
        ],
    )
    def body(s3_hbm, w_hbm, smask_hbm, sig_hbm,
             row0, row1, row2, cmb, cand_a, cand_b, wv):
        wid = lax.axis_index("s") * info.num_cores + lax.axis_index("c")
        pltpu.sync_copy(w_hbm, wv)
        w0 = _sc_rne(wv[pl.ds(0, 16)])
        w1 = _sc_rne(wv[pl.ds(16, 16)])
        w2 = _sc_rne(wv[pl.ds(32, 16)])
        wb = wv[pl.ds(48, 16)]
        for r in range(rows_per):
            b = wid * rows_per + r
            pltpu.sync_copy(s3_hbm.at[0, b], row0)
            pltpu.sync_copy(s3_hbm.at[1, b], row1)
            pltpu.sync_copy(s3_hbm.at[2, b], row2)
            for rr in (row0, row1, row2):
                _sc_mask_row(rr, cand_a, cand_b, k, T)

            def mix(j, acc):
                for u in range(4):
                    o = (j * 4 + u) * 16
                    m0 = _sc_rne(row0[pl.ds(o, 16)])
                    m1 = _sc_rne(row1[pl.ds(o, 16)])
                    m2 = _sc_rne(row2[pl.ds(o, 16)])
                    cmb[pl.ds(o, 16)] = m0 * w0 + m1 * w1 + m2 * w2 + wb
                return acc

            lax.fori_loop(0, T // 64, mix, jnp.int32(0))
            _sc_mask_row(cmb, cand_a, cand_b, k, T)
            pltpu.sync_copy(cmb, smask_hbm.at[b])

            def sg(j, acc):
                for u in range(4):
                    o = (j * 4 + u) * 16
                    x = cmb[pl.ds(o, 16)]
                    row0[pl.ds(o, 16)] = 1.0 / (1.0 + jnp.exp(-x))
                return acc

            lax.fori_loop(0, T // 64, sg, jnp.int32(0))
            pltpu.sync_copy(row0, sig_hbm.at[b])

    return body(scores3, wsplat)


def _finalize_body(values_ref, sig_ref, ctx_ref, att_ref):
    sg = sig_ref[...]
    colsum = jnp.sum(sg, axis=0, keepdims=True)
    att = sg / colsum
    att_ref[...] = att
    v = values_ref[...]
    partial = jnp.sum(att[:, :, None] * v, axis=1)

    @pl.when(pl.program_id(0) == 0)
    def _():
        ctx_ref[...] = jnp.zeros_like(ctx_ref)

    ctx_ref[...] += partial


def _finalize_call(values, sig):
    B, T, H = values.shape
    Tb = 128 if T % 128 == 0 else T
    return pl.pallas_call(
        _finalize_body,
        grid=(T // Tb,),
        in_specs=[
            pl.BlockSpec((B, Tb, H), lambda i: (0, i, 0)),
            pl.BlockSpec((B, Tb), lambda i: (0, i)),
        ],
        out_specs=[
            pl.BlockSpec((B, H), lambda i: (0, 0)),
            pl.BlockSpec((B, Tb), lambda i: (0, i)),
        ],
        out_shape=[
            jax.ShapeDtypeStruct((B, H), jnp.float32),
            jax.ShapeDtypeStruct((B, T), jnp.float32),
        ],
    )(values, sig)


def kernel(query, values, prev_att, params):
    B, T, H = values.shape
    heads = params['heads']
    mid = heads[0]['conv_w'].shape[-1] // 2
    k = T * 2 // 3

    prev2 = prev_att[..., 0]
    cw3 = jnp.stack([hp['conv_w'][:, :, mid] for hp in heads])
    w1t = jnp.stack([hp['W1_w'].T for hp in heads])
    w2t = jnp.stack([hp['W2_w'].T for hp in heads])
    qb = jnp.stack([hp['W1_b'] + hp['W2_b'] for hp in heads])
    locp = jnp.stack([hp['loc_proj_w'][:, 0] for hp in heads])
    vw = jnp.stack([hp['V_w'][0] for hp in heads])
    vb = jnp.stack([hp['V_b'][0] for hp in heads])
    ws = [jnp.full((16,), params['W_w'][0, i], jnp.float32) for i in range(3)]
    wsplat = jnp.concatenate(ws + [jnp.full((16,), params['W_b'][0],
                                            jnp.float32)])

    scores3 = _scores_call(values, cw3, prev2, query, w1t, w2t, qb, locp,
                           vw, vb)
    smask, sig = _sc_mask_call(scores3, wsplat, k)
    ctx, att = _finalize_call(values, sig)
    return ctx, att[..., None], smask[..., None]


# speedup vs baseline: 1.0619x; 1.0542x over previous
import functools

import jax
import jax.numpy as jnp
from jax import lax
from jax.experimental import pallas as pl
from jax.experimental.pallas import tpu as pltpu
from jax.experimental.pallas import tpu_sc as plsc


def _bf16_rne(x):
    u = lax.bitcast_convert_type(x, jnp.int32)
    r = (u + jnp.int32(0x7FFF) + ((u >> 16) & 1)) & jnp.int32(-65536)
    return lax.bitcast_convert_type(r, jnp.float32)


def _sortable(x):
    xi = lax.bitcast_convert_type(x, jnp.int32)
    return xi ^ ((xi >> 31) & jnp.int32(0x7FFFFFFF))


def _topk_mask(v, s, k):
    tau = _kth_largest(v, k)
    gt = v > tau
    eq = v == tau
    need = k - jnp.sum(gt.astype(jnp.int32), axis=-1, keepdims=True)
    N = v.shape[-1]
    iota = lax.broadcasted_iota(jnp.int32, v.shape, v.ndim - 1)
    eqi = eq.astype(jnp.int32)
    c = jnp.zeros(v.shape[:-1] + (1,), jnp.int32)
    b = N // 2
    while b >= 1:
        cnt_lt = jnp.sum(jnp.where(iota < (c + b), eqi, 0),
                         axis=-1, keepdims=True)
        c = jnp.where(cnt_lt < need, c + b, c)
        b //= 2
    keep = gt | (eq & (iota <= c))
    return jnp.where(keep, s, 0.0)


def _kth_largest(v, k):
    p0 = jnp.full(v.shape[:-1] + (1,), jnp.int32(-(2**31)))

    def body(i, p):
        c = p + (jnp.int32(1) << (jnp.int32(31) - i))
        cnt = jnp.sum((v >= c).astype(jnp.int32), axis=-1, keepdims=True)
        return jnp.where(cnt >= k, c, p)

    return lax.fori_loop(0, 32, body, p0)


def _scores_body(values_ref, cw3_ref, prev_ref, query_ref, w1t_ref, w2t_ref,
                 qb_ref, locp_ref, vw_ref, vb_ref, out_ref):
    B, Tb, H = values_ref.shape
    prev = prev_ref[...]
    q = query_ref[...]
    Rb = min(16, B)
    for h in range(3):
        cw = cw3_ref[h]
        convo = lax.dot_general(prev, cw, (((1,), (1,)), ((), ())),
                                preferred_element_type=jnp.float32)
        qt = jnp.dot(q, w2t_ref[h],
                     preferred_element_type=jnp.float32) + qb_ref[h][None, :]
        vwb = jnp.broadcast_to(vw_ref[h][:, None], (H, 128))
        for rb in range(0, B, Rb):
            v = values_ref[rb:rb + Rb].reshape(Rb * Tb, H)
            p1 = jnp.dot(v, w1t_ref[h],
                         preferred_element_type=jnp.float32).reshape(Rb, Tb, H)
            s1 = (p1 + qt[rb:rb + Rb, None, :]
                  + convo[rb:rb + Rb, :, None] * locp_ref[h][None, None, :])
            z = jnp.tanh(s1).reshape(Rb * Tb, H)
            sc = jnp.dot(z, vwb,
                         preferred_element_type=jnp.float32)
            sc = sc.reshape(Rb, Tb, 128)[:, :, 0] + vb_ref[h]
            out_ref[h, rb:rb + Rb] = sc


def _scores_call(values, cw3, prev2, query, w1t, w2t, qb, locp, vw, vb):
    B, T, H = values.shape
    Tb = 128 if T % 128 == 0 else T
    return pl.pallas_call(
        _scores_body,
        grid=(T // Tb,),
        in_specs=[
            pl.BlockSpec((B, Tb, H), lambda i: (0, i, 0)),
            pl.BlockSpec((3, Tb, T), lambda i: (0, i, 0)),
            pl.BlockSpec((B, T), lambda i: (0, 0)),
            pl.BlockSpec((B, H), lambda i: (0, 0)),
            pl.BlockSpec((3, H, H), lambda i: (0, 0, 0)),
            pl.BlockSpec((3, H, H), lambda i: (0, 0, 0)),
            pl.BlockSpec((3, H), lambda i: (0, 0)),
            pl.BlockSpec((3, H), lambda i: (0, 0)),
            pl.BlockSpec((3, H), lambda i: (0, 0)),
            pl.BlockSpec(memory_space=pltpu.SMEM),
        ],
        out_specs=pl.BlockSpec((3, B, Tb), lambda i: (0, 0, i)),
        out_shape=jax.ShapeDtypeStruct((3, B, T), jnp.float32),
    )(values, cw3, prev2, query, w1t, w2t, qb, locp, vw, vb)


def _mask_body(s3_ref, wmix_ref, smask_ref, sig_ref, *, k):
    s3 = s3_ref[...]
    v3 = _sortable(s3)
    m3 = _topk_mask(v3, s3, k)
    m3b = _bf16_rne(m3)
    w0 = _bf16_rne(wmix_ref[0])
    w1 = _bf16_rne(wmix_ref[1])
    w2 = _bf16_rne(wmix_ref[2])
    cmb = m3b[0] * w0 + m3b[1] * w1 + m3b[2] * w2 + wmix_ref[3]
    vc = _sortable(cmb)
    sm = _topk_mask(vc, cmb, k)
    smask_ref[...] = sm
    sig_ref[...] = 1.0 / (1.0 + jnp.exp(-sm))


def _mask_call(scores3, wmix4, k):
    import functools
    _, B, T = scores3.shape
    return pl.pallas_call(
        functools.partial(_mask_body, k=k),
        in_specs=[
            pl.BlockSpec((3, B, T), lambda: (0, 0, 0)),
            pl.BlockSpec(memory_space=pltpu.SMEM),
        ],
        out_specs=[
            pl.BlockSpec((B, T), lambda: (0, 0)),
            pl.BlockSpec((B, T), lambda: (0, 0)),
        ],
        out_shape=[
            jax.ShapeDtypeStruct((B, T), jnp.float32),
            jax.ShapeDtypeStruct((B, T), jnp.float32),
        ],
    )(scores3, wmix4)


def _popc(m):
    return plsc.all_reduce_population_count(m)[0]


def _sc_sortable(v):
    u = lax.bitcast_convert_type(v, jnp.int32)
    return u ^ ((u >> 31) & jnp.int32(0x7FFFFFFF))


def _sc_mask_row(row_ref, cand_a, cand_b, k, T):
    U = 4
    iota16 = lax.iota(jnp.int32, 16)

    def kb(j, acc):
        for u in range(U):
            v = row_ref[pl.ds((j * U + u) * 16, 16)]
            cand_a[pl.ds((j * U + u) * 16, 16)] = _sc_sortable(v)
        return acc

    lax.fori_loop(0, T // (16 * U), kb, jnp.int32(0))

    def half_step(i2, src, dst, p, rem, n):
        c = p + (jnp.int32(1) << (jnp.int32(31) - i2))
        nch = (n + (16 * U - 1)) // (16 * U)

        def cb(j, acc):
            for u in range(U):
                v = src[pl.ds((j * U + u) * 16, 16)]
                valid = ((j * U + u) * 16 + iota16) < n
                m = (v >= c) & valid
                acc = acc + _popc(m)
            return acc

        cnt = lax.fori_loop(0, nch, cb, jnp.int32(0))
        takehi = cnt >= rem
        want = jnp.where(takehi, jnp.int32(1), jnp.int32(0))

        def pb(j, off):
            for u in range(U):
                v = src[pl.ds((j * U + u) * 16, 16)]
                valid = ((j * U + u) * 16 + iota16) < n
                m = (jnp.where(v >= c, 1, 0) == want) & valid
                plsc.store_compressed(dst.at[pl.ds(off, 16)], v, mask=m)
                off = off + _popc(m)
            return off

        lax.fori_loop(0, nch, pb, jnp.int32(0))
        p = jnp.where(takehi, c, p)
        rem = jnp.where(takehi, rem, rem - cnt)
        n = jnp.where(takehi, cnt, n - cnt)
        return p, rem, n

    def bit_pair(i, carry):
        p, rem, n = carry
        p, rem, n = half_step(2 * i, cand_a, cand_b, p, rem, n)
        p, rem, n = half_step(2 * i + 1, cand_b, cand_a, p, rem, n)
        return p, rem, n

    p, rem, _ = lax.fori_loop(
        0, 16, bit_pair,
        (jnp.int32(-(2**31)), jnp.int32(k), jnp.int32(T)))
    tau = p

    def fb(j, run):
        for u in range(U):
            v = row_ref[pl.ds((j * U + u) * 16, 16)]
            uu = _sc_sortable(v)
            m_gt = uu > tau
            m_eq = uu == tau
            eqi = jnp.where(m_eq, 1, 0)
            pc = plsc.cumsum(eqi)
            keep = m_gt | (m_eq & ((run + pc) <= rem))
            row_ref[pl.ds((j * U + u) * 16, 16)] = jnp.where(keep, v, 0.0)
            run = run + _popc(m_eq)
        return run

    lax.fori_loop(0, T // (16 * U), fb, jnp.int32(0))


def _sc_rne(x):
    u = lax.bitcast_convert_type(x, jnp.int32)
    r = (u + jnp.int32(0x7FFF) + ((u >> 16) & 1)) & jnp.int32(-65536)
    return lax.bitcast_convert_type(r, jnp.float32)


def _sc_mask_call(scores3, wsplat, k):
    _, B, T = scores3.shape
    info = plsc.get_sparse_core_info()
    nw = info.num_cores * info.num_subcores
    rows_per = B // nw

    @functools.partial(
        pl.kernel,
        mesh=plsc.VectorSubcoreMesh(core_axis_name="c", subcore_axis_name="s"),
        compiler_params=pltpu.CompilerParams(needs_layout_passes=False),
        out_type=[
            jax.ShapeDtypeStruct((B, T), jnp.float32),
            jax.ShapeDtypeStruct((B, T), jnp.float32),
        ],
        scratch_types=[
            pltpu.VMEM((T,), jnp.float32),
            pltpu.VMEM((T,), jnp.float32),
            pltpu.VMEM((T,), jnp.float32),
            pltpu.VMEM((T,), jnp.float32),
            pltpu.VMEM((T + 16,), jnp.int32),
            pltpu.VMEM((T + 16,), jnp.int32),
            pltpu.VMEM((64,), jnp.float32),
        ],
    )
    def body(s3_hbm, w_hbm, smask_hbm, sig_hbm,
             row0, row1, row2, cmb, cand_a, cand_b, wv):
        wid = lax.axis_index("s") * info.num_cores + lax.axis_index("c")
        pltpu.sync_copy(w_hbm, wv)
        w0 = _sc_rne(wv[pl.ds(0, 16)])
        w1 = _sc_rne(wv[pl.ds(16, 16)])
        w2 = _sc_rne(wv[pl.ds(32, 16)])
        wb = wv[pl.ds(48, 16)]
        for r in range(rows_per):
            b = wid * rows_per + r
            pltpu.sync_copy(s3_hbm.at[0, b], row0)
            pltpu.sync_copy(s3_hbm.at[1, b], row1)
            pltpu.sync_copy(s3_hbm.at[2, b], row2)
            for rr in (row0, row1, row2):
                _sc_mask_row(rr, cand_a, cand_b, k, T)

            def mix(j, acc):
                for u in range(4):
                    o = (j * 4 + u) * 16
                    m0 = _sc_rne(row0[pl.ds(o, 16)])
                    m1 = _sc_rne(row1[pl.ds(o, 16)])
                    m2 = _sc_rne(row2[pl.ds(o, 16)])
                    cmb[pl.ds(o, 16)] = m0 * w0 + m1 * w1 + m2 * w2 + wb
                return acc

            lax.fori_loop(0, T // 64, mix, jnp.int32(0))
            _sc_mask_row(cmb, cand_a, cand_b, k, T)
            pltpu.sync_copy(cmb, smask_hbm.at[b])

            def sg(j, acc):
                for u in range(4):
                    o = (j * 4 + u) * 16
                    x = cmb[pl.ds(o, 16)]
                    row0[pl.ds(o, 16)] = 1.0 / (1.0 + jnp.exp(-x))
                return acc

            lax.fori_loop(0, T // 64, sg, jnp.int32(0))
            pltpu.sync_copy(row0, sig_hbm.at[b])

    return body(scores3, wsplat)


def _finalize_body(values_ref, sig_ref, ctx_ref, att_ref):
    sg = sig_ref[...]
    colsum = jnp.sum(sg, axis=0, keepdims=True)
    att = sg / colsum
    att_ref[...] = att
    v = values_ref[...]
    partial = jnp.sum(att[:, :, None] * v, axis=1)

    @pl.when(pl.program_id(0) == 0)
    def _():
        ctx_ref[...] = jnp.zeros_like(ctx_ref)

    ctx_ref[...] += partial


def _finalize_call(values, sig):
    B, T, H = values.shape
    Tb = 128 if T % 128 == 0 else T
    return pl.pallas_call(
        _finalize_body,
        grid=(T // Tb,),
        in_specs=[
            pl.BlockSpec((B, Tb, H), lambda i: (0, i, 0)),
            pl.BlockSpec((B, Tb), lambda i: (0, i)),
        ],
        out_specs=[
            pl.BlockSpec((B, H), lambda i: (0, 0)),
            pl.BlockSpec((B, Tb), lambda i: (0, i)),
        ],
        out_shape=[
            jax.ShapeDtypeStruct((B, H), jnp.float32),
            jax.ShapeDtypeStruct((B, T), jnp.float32),
        ],
    )(values, sig)


def kernel(query, values, prev_att, params):
    B, T, H = values.shape
    heads = params['heads']
    mid = heads[0]['conv_w'].shape[-1] // 2
    k = T * 2 // 3

    prev2 = prev_att[..., 0]
    cw3 = jnp.stack([hp['conv_w'][:, :, mid] for hp in heads])
    w1t = jnp.stack([hp['W1_w'].T for hp in heads])
    w2t = jnp.stack([hp['W2_w'].T for hp in heads])
    qb = jnp.stack([hp['W1_b'] + hp['W2_b'] for hp in heads])
    locp = jnp.stack([hp['loc_proj_w'][:, 0] for hp in heads])
    vw = jnp.stack([hp['V_w'][0] for hp in heads])
    vb = jnp.stack([hp['V_b'][0] for hp in heads])
    ws = [jnp.full((16,), params['W_w'][0, i], jnp.float32) for i in range(3)]
    wsplat = jnp.concatenate(ws + [jnp.full((16,), params['W_b'][0],
                                            jnp.float32)])

    scores3 = _scores_call(values, cw3, prev2, query, w1t, w2t, qb, locp,
                           vw, vb)
    smask, sig = _sc_mask_call(scores3, wsplat, k)
    ctx, att = _finalize_call(values, sig)
    return ctx, att[..., None], smask[..., None]


# speedup vs baseline: 1.0966x; 1.0327x over previous
import functools

import jax
import jax.numpy as jnp
from jax import lax
from jax.experimental import pallas as pl
from jax.experimental.pallas import tpu as pltpu
from jax.experimental.pallas import tpu_sc as plsc


def _bf16_rne(x):
    u = lax.bitcast_convert_type(x, jnp.int32)
    r = (u + jnp.int32(0x7FFF) + ((u >> 16) & 1)) & jnp.int32(-65536)
    return lax.bitcast_convert_type(r, jnp.float32)


def _sortable(x):
    xi = lax.bitcast_convert_type(x, jnp.int32)
    return xi ^ ((xi >> 31) & jnp.int32(0x7FFFFFFF))


def _topk_mask(v, s, k):
    tau = _kth_largest(v, k)
    gt = v > tau
    eq = v == tau
    need = k - jnp.sum(gt.astype(jnp.int32), axis=-1, keepdims=True)
    N = v.shape[-1]
    iota = lax.broadcasted_iota(jnp.int32, v.shape, v.ndim - 1)
    eqi = eq.astype(jnp.int32)
    c = jnp.zeros(v.shape[:-1] + (1,), jnp.int32)
    b = N // 2
    while b >= 1:
        cnt_lt = jnp.sum(jnp.where(iota < (c + b), eqi, 0),
                         axis=-1, keepdims=True)
        c = jnp.where(cnt_lt < need, c + b, c)
        b //= 2
    keep = gt | (eq & (iota <= c))
    return jnp.where(keep, s, 0.0)


def _kth_largest(v, k):
    p0 = jnp.full(v.shape[:-1] + (1,), jnp.int32(-(2**31)))

    def body(i, p):
        c = p + (jnp.int32(1) << (jnp.int32(31) - i))
        cnt = jnp.sum((v >= c).astype(jnp.int32), axis=-1, keepdims=True)
        return jnp.where(cnt >= k, c, p)

    return lax.fori_loop(0, 32, body, p0)


def _scores_body(values_ref, cw3_ref, prev_ref, query_ref, w1t_ref, w2t_ref,
                 qb_ref, locp_ref, vw_ref, vb_ref, out_ref):
    B, Tb, H = values_ref.shape
    prev = prev_ref[...]
    q = query_ref[...]
    Rb = min(16, B)
    for h in range(3):
        cw = cw3_ref[h]
        convo = lax.dot_general(prev, cw, (((1,), (1,)), ((), ())),
                                preferred_element_type=jnp.float32)
        qt = jnp.dot(q, w2t_ref[h],
                     preferred_element_type=jnp.float32) + qb_ref[h][None, :]
        vwb = jnp.broadcast_to(vw_ref[h][:, None], (H, 128))
        for rb in range(0, B, Rb):
            v = values_ref[rb:rb + Rb].reshape(Rb * Tb, H)
            p1 = jnp.dot(v, w1t_ref[h],
                         preferred_element_type=jnp.float32).reshape(Rb, Tb, H)
            s1 = (p1 + qt[rb:rb + Rb, None, :]
                  + convo[rb:rb + Rb, :, None] * locp_ref[h][None, None, :])
            z = jnp.tanh(s1).reshape(Rb * Tb, H)
            sc = jnp.dot(z, vwb,
                         preferred_element_type=jnp.float32)
            sc = sc.reshape(Rb, Tb, 128)[:, :, 0] + vb_ref[h]
            out_ref[h, rb:rb + Rb] = sc


def _scores_call(values, cw3, prev2, query, w1t, w2t, qb, locp, vw, vb):
    B, T, H = values.shape
    Tb = 128 if T % 128 == 0 else T
    return pl.pallas_call(
        _scores_body,
        grid=(T // Tb,),
        in_specs=[
            pl.BlockSpec((B, Tb, H), lambda i: (0, i, 0)),
            pl.BlockSpec((3, Tb, T), lambda i: (0, i, 0)),
            pl.BlockSpec((B, T), lambda i: (0, 0)),
            pl.BlockSpec((B, H), lambda i: (0, 0)),
            pl.BlockSpec((3, H, H), lambda i: (0, 0, 0)),
            pl.BlockSpec((3, H, H), lambda i: (0, 0, 0)),
            pl.BlockSpec((3, H), lambda i: (0, 0)),
            pl.BlockSpec((3, H), lambda i: (0, 0)),
            pl.BlockSpec((3, H), lambda i: (0, 0)),
            pl.BlockSpec(memory_space=pltpu.SMEM),
        ],
        out_specs=pl.BlockSpec((3, B, Tb), lambda i: (0, 0, i)),
        out_shape=jax.ShapeDtypeStruct((3, B, T), jnp.float32),
    )(values, cw3, prev2, query, w1t, w2t, qb, locp, vw, vb)


def _mask_body(s3_ref, wmix_ref, smask_ref, sig_ref, *, k):
    s3 = s3_ref[...]
    v3 = _sortable(s3)
    m3 = _topk_mask(v3, s3, k)
    m3b = _bf16_rne(m3)
    w0 = _bf16_rne(wmix_ref[0])
    w1 = _bf16_rne(wmix_ref[1])
    w2 = _bf16_rne(wmix_ref[2])
    cmb = m3b[0] * w0 + m3b[1] * w1 + m3b[2] * w2 + wmix_ref[3]
    vc = _sortable(cmb)
    sm = _topk_mask(vc, cmb, k)
    smask_ref[...] = sm
    sig_ref[...] = 1.0 / (1.0 + jnp.exp(-sm))


def _mask_call(scores3, wmix4, k):
    import functools
    _, B, T = scores3.shape
    return pl.pallas_call(
        functools.partial(_mask_body, k=k),
        in_specs=[
            pl.BlockSpec((3, B, T), lambda: (0, 0, 0)),
            pl.BlockSpec(memory_space=pltpu.SMEM),
        ],
        out_specs=[
            pl.BlockSpec((B, T), lambda: (0, 0)),
            pl.BlockSpec((B, T), lambda: (0, 0)),
        ],
        out_shape=[
            jax.ShapeDtypeStruct((B, T), jnp.float32),
            jax.ShapeDtypeStruct((B, T), jnp.float32),
        ],
    )(scores3, wmix4)


def _popc(m):
    return plsc.all_reduce_population_count(m)[0]


def _sc_sortable(v):
    u = lax.bitcast_convert_type(v, jnp.int32)
    return u ^ ((u >> 31) & jnp.int32(0x7FFFFFFF))


def _sc_mask_row(row_ref, cand_a, cand_b, k, T):
    U = 4
    iota16 = lax.iota(jnp.int32, 16)

    def kb(j, acc):
        for u in range(U):
            v = row_ref[pl.ds((j * U + u) * 16, 16)]
            cand_a[pl.ds((j * U + u) * 16, 16)] = _sc_sortable(v)
        return acc

    lax.fori_loop(0, T // (16 * U), kb, jnp.int32(0))

    CB = T + 16

    def half_step(i2, src, dst, base, p, rem, n):
        c = p + (jnp.int32(1) << (jnp.int32(31) - i2))
        nch = (n + (16 * U - 1)) // (16 * U)

        def pb(j, carry):
            offs, offc = carry
            for u in range(U):
                v = src[pl.ds(base + (j * U + u) * 16, 16)]
                valid = ((j * U + u) * 16 + iota16) < n
                hi = v >= c
                mset = hi & valid
                mclr = (~hi) & valid
                plsc.store_compressed(dst.at[pl.ds(offs, 16)], v, mask=mset)
                plsc.store_compressed(dst.at[pl.ds(offc, 16)], v, mask=mclr)
                offs = offs + _popc(mset)
                offc = offc + _popc(mclr)
            return offs, offc

        offs, offc = lax.fori_loop(0, nch, pb,
                                   (jnp.int32(0), jnp.int32(CB)))
        cnt = offs
        takehi = cnt >= rem
        p = jnp.where(takehi, c, p)
        rem = jnp.where(takehi, rem, rem - cnt)
        n = jnp.where(takehi, cnt, n - cnt)
        base = jnp.where(takehi, jnp.int32(0), jnp.int32(CB))
        return base, p, rem, n

    def bit_pair(i, carry):
        ba, p, rem, n = carry
        bb, p, rem, n = half_step(2 * i, cand_a, cand_b, ba, p, rem, n)
        ba, p, rem, n = half_step(2 * i + 1, cand_b, cand_a, bb, p, rem, n)
        return ba, p, rem, n

    _, p, rem, _ = lax.fori_loop(
        0, 16, bit_pair,
        (jnp.int32(0), jnp.int32(-(2**31)), jnp.int32(k), jnp.int32(T)))
    tau = p

    def fb(j, run):
        for u in range(U):
            v = row_ref[pl.ds((j * U + u) * 16, 16)]
            uu = _sc_sortable(v)
            m_gt = uu > tau
            m_eq = uu == tau
            eqi = jnp.where(m_eq, 1, 0)
            pc = plsc.cumsum(eqi)
            keep = m_gt | (m_eq & ((run + pc) <= rem))
            row_ref[pl.ds((j * U + u) * 16, 16)] = jnp.where(keep, v, 0.0)
            run = run + _popc(m_eq)
        return run

    lax.fori_loop(0, T // (16 * U), fb, jnp.int32(0))


def _sc_rne(x):
    u = lax.bitcast_convert_type(x, jnp.int32)
    r = (u + jnp.int32(0x7FFF) + ((u >> 16) & 1)) & jnp.int32(-65536)
    return lax.bitcast_convert_type(r, jnp.float32)


def _sc_mask_call(scores3, wsplat, k):
    _, B, T = scores3.shape
    info = plsc.get_sparse_core_info()
    nw = info.num_cores * info.num_subcores
    rows_per = B // nw

    @functools.partial(
        pl.kernel,
        mesh=plsc.VectorSubcoreMesh(core_axis_name="c", subcore_axis_name="s"),
        compiler_params=pltpu.CompilerParams(needs_layout_passes=False),
        out_type=[
            jax.ShapeDtypeStruct((B, T), jnp.float32),
            jax.ShapeDtypeStruct((B, T), jnp.float32),
        ],
        scratch_types=[
            pltpu.VMEM((T,), jnp.float32),
            pltpu.VMEM((T,), jnp.float32),
            pltpu.VMEM((T,), jnp.float32),
            pltpu.VMEM((T,), jnp.float32),
            pltpu.VMEM((2 * T + 32,), jnp.int32),
            pltpu.VMEM((2 * T + 32,), jnp.int32),
            pltpu.VMEM((64,), jnp.float32),
        ],
    )
    def body(s3_hbm, w_hbm, smask_hbm, sig_hbm,
             row0, row1, row2, cmb, cand_a, cand_b, wv):
        wid = lax.axis_index("s") * info.num_cores + lax.axis_index("c")
        pltpu.sync_copy(w_hbm, wv)
        w0 = _sc_rne(wv[pl.ds(0, 16)])
        w1 = _sc_rne(wv[pl.ds(16, 16)])
        w2 = _sc_rne(wv[pl.ds(32, 16)])
        wb = wv[pl.ds(48, 16)]
        for r in range(rows_per):
            b = wid * rows_per + r
            pltpu.sync_copy(s3_hbm.at[0, b], row0)
            pltpu.sync_copy(s3_hbm.at[1, b], row1)
            pltpu.sync_copy(s3_hbm.at[2, b], row2)
            for rr in (row0, row1, row2):
                _sc_mask_row(rr, cand_a, cand_b, k, T)

            def mix(j, acc):
                for u in range(4):
                    o = (j * 4 + u) * 16
                    m0 = _sc_rne(row0[pl.ds(o, 16)])
                    m1 = _sc_rne(row1[pl.ds(o, 16)])
                    m2 = _sc_rne(row2[pl.ds(o, 16)])
                    cmb[pl.ds(o, 16)] = m0 * w0 + m1 * w1 + m2 * w2 + wb
                return acc

            lax.fori_loop(0, T // 64, mix, jnp.int32(0))
            _sc_mask_row(cmb, cand_a, cand_b, k, T)
            pltpu.sync_copy(cmb, smask_hbm.at[b])

            def sg(j, acc):
                for u in range(4):
                    o = (j * 4 + u) * 16
                    x = cmb[pl.ds(o, 16)]
                    row0[pl.ds(o, 16)] = 1.0 / (1.0 + jnp.exp(-x))
                return acc

            lax.fori_loop(0, T // 64, sg, jnp.int32(0))
            pltpu.sync_copy(row0, sig_hbm.at[b])

    return body(scores3, wsplat)


def _finalize_body(values_ref, sig_ref, ctx_ref, att_ref):
    sg = sig_ref[...]
    colsum = jnp.sum(sg, axis=0, keepdims=True)
    att = sg / colsum
    att_ref[...] = att
    v = values_ref[...]
    partial = jnp.sum(att[:, :, None] * v, axis=1)

    @pl.when(pl.program_id(0) == 0)
    def _():
        ctx_ref[...] = jnp.zeros_like(ctx_ref)

    ctx_ref[...] += partial


def _finalize_call(values, sig):
    B, T, H = values.shape
    Tb = 128 if T % 128 == 0 else T
    return pl.pallas_call(
        _finalize_body,
        grid=(T // Tb,),
        in_specs=[
            pl.BlockSpec((B, Tb, H), lambda i: (0, i, 0)),
            pl.BlockSpec((B, Tb), lambda i: (0, i)),
        ],
        out_specs=[
            pl.BlockSpec((B, H), lambda i: (0, 0)),
            pl.BlockSpec((B, Tb), lambda i: (0, i)),
        ],
        out_shape=[
            jax.ShapeDtypeStruct((B, H), jnp.float32),
            jax.ShapeDtypeStruct((B, T), jnp.float32),
        ],
    )(values, sig)


def kernel(query, values, prev_att, params):
    B, T, H = values.shape
    heads = params['heads']
    mid = heads[0]['conv_w'].shape[-1] // 2
    k = T * 2 // 3

    prev2 = prev_att[..., 0]
    cw3 = jnp.stack([hp['conv_w'][:, :, mid] for hp in heads])
    w1t = jnp.stack([hp['W1_w'].T for hp in heads])
    w2t = jnp.stack([hp['W2_w'].T for hp in heads])
    qb = jnp.stack([hp['W1_b'] + hp['W2_b'] for hp in heads])
    locp = jnp.stack([hp['loc_proj_w'][:, 0] for hp in heads])
    vw = jnp.stack([hp['V_w'][0] for hp in heads])
    vb = jnp.stack([hp['V_b'][0] for hp in heads])
    ws = [jnp.full((16,), params['W_w'][0, i], jnp.float32) for i in range(3)]
    wsplat = jnp.concatenate(ws + [jnp.full((16,), params['W_b'][0],
                                            jnp.float32)])

    scores3 = _scores_call(values, cw3, prev2, query, w1t, w2t, qb, locp,
                           vw, vb)
    smask, sig = _sc_mask_call(scores3, wsplat, k)
    ctx, att = _finalize_call(values, sig)
    return ctx, att[..., None], smask[..., None]
